# baseline (speedup 1.0000x reference)
import jax
import jax.numpy as jnp
from jax import lax
from jax.experimental import pallas as pl
from jax.experimental.pallas import tpu as pltpu
from jax.experimental.pallas import tpu_sc as plsc

D_MODEL = 768
N_EXPERTS = 8
TOP_K = 2
BALANCE_LOSS_WEIGHT = 0.01
CAPACITY_FACTOR = 1.25

BM = 4096
N_SC = 2048
NUM_TILES = 32
LANES = 16
NT = N_SC // NUM_TILES
CHUNK = 16
KCH = D_MODEL // LANES


def _tc_body(wt_ref, x_ref, logits_ref, idx_ref, prob_ref, part_ref, n_tc):
    i = pl.program_id(0)
    wt = wt_ref[...]
    logits = jnp.dot(x_ref[...], wt, preferred_element_type=jnp.float32)
    lt = logits.T
    logits_ref[...] = lt

    iota = lax.broadcasted_iota(jnp.int32, (N_EXPERTS, BM), 0)
    m1 = jnp.max(lt, axis=0, keepdims=True)
    i1 = jnp.min(jnp.where(lt == m1, iota, N_EXPERTS), axis=0, keepdims=True)
    masked = jnp.where(iota == i1, -jnp.inf, lt)
    m2 = jnp.max(masked, axis=0, keepdims=True)
    i2 = jnp.min(jnp.where(masked == m2, iota, N_EXPERTS), axis=0, keepdims=True)

    t = jnp.exp(m2 - m1)
    denom2 = 1.0 + t
    idx_ref[...] = jnp.concatenate([i1, i2], axis=0)
    prob_ref[...] = jnp.concatenate([1.0 / denom2, t / denom2], axis=0)

    tid = i * BM + lax.broadcasted_iota(jnp.int32, (N_EXPERTS, BM), 1)
    valid = tid < n_tc
    e = jnp.exp(lt - m1)
    gp = jnp.where(valid, e / jnp.sum(e, axis=0, keepdims=True), 0.0)
    ps_blk = jnp.sum(gp, axis=1, keepdims=True)
    one = jnp.where(valid, 1.0, 0.0)
    cnt_blk = (jnp.sum(jnp.where(iota == i1, one, 0.0), axis=1, keepdims=True)
               + jnp.sum(jnp.where(iota == i2, one, 0.0), axis=1, keepdims=True))

    part_ref[...] = jnp.concatenate([cnt_blk, ps_blk], axis=1)[None]


def _tc_router(x_flat, wt, n_tc):
    nsteps = (n_tc + BM - 1) // BM
    body = lambda *refs: _tc_body(*refs, n_tc)
    return pl.pallas_call(
        body,
        grid=(nsteps,),
        in_specs=[
            pl.BlockSpec((D_MODEL, N_EXPERTS), lambda i: (0, 0)),
            pl.BlockSpec((BM, D_MODEL), lambda i: (i, 0)),
        ],
        out_specs=[
            pl.BlockSpec((N_EXPERTS, BM), lambda i: (0, i)),
            pl.BlockSpec((TOP_K, BM), lambda i: (0, i)),
            pl.BlockSpec((TOP_K, BM), lambda i: (0, i)),
            pl.BlockSpec((1, N_EXPERTS, 2), lambda i: (i, 0, 0)),
        ],
        out_shape=[
            jax.ShapeDtypeStruct((N_EXPERTS, n_tc), jnp.float32),
            jax.ShapeDtypeStruct((TOP_K, n_tc), jnp.int32),
            jax.ShapeDtypeStruct((TOP_K, n_tc), jnp.float32),
            jax.ShapeDtypeStruct((nsteps, N_EXPERTS, 2), jnp.float32),
        ],
        compiler_params=pltpu.CompilerParams(
            dimension_semantics=("parallel",),
        ),
    )(wt, x_flat)


def _perm(v, idx):
    return lax.gather(
        v, idx.reshape(LANES, 1),
        lax.GatherDimensionNumbers(offset_dims=(), collapsed_slice_dims=(0,),
                                   start_index_map=(0,)),
        (1,), mode=lax.GatherScatterMode.PROMISE_IN_BOUNDS)


def _sc_body(x_hbm, w_hbm, lg_hbm, i1_hbm, i2_hbm, p1_hbm, p2_hbm, part_hbm,
             w_v, x_v, lg_v, i1_v, i2_v, p1_v, p2_v, acc_v):
    wid = lax.axis_index("s") * 2 + lax.axis_index("c")
    sc_base = x_hbm.shape[0] - N_SC
    row0 = sc_base + wid * NT

    pltpu.sync_copy(w_hbm, w_v)

    iota = lax.iota(jnp.int32, LANES)
    iota8 = lax.bitwise_and(iota, 7)
    cperm = lax.bitwise_and(iota, 1) * 8
    zero = jnp.zeros((LANES,), jnp.float32)
    neg_inf = jnp.full((LANES,), -jnp.inf, jnp.float32)

    acc_v[0, :] = zero
    acc_v[1, :] = zero

    def butterfly_sum16(vs):
        d = 1
        while len(vs) > 1:
            bit = lax.bitwise_and(iota, d) == 0
            pidx = lax.bitwise_xor(iota, d)
            nxt = []
            for j in range(0, len(vs), 2):
                a, b = vs[j], vs[j + 1]
                s = jnp.where(bit, a, b)
                p = jnp.where(bit, _perm(a, pidx), _perm(b, pidx))
                nxt.append(s + p)
            vs = nxt
            d *= 2
        return vs[0]

    def half_sum(v):
        for d in (1, 2, 4):
            v = v + _perm(v, lax.bitwise_xor(iota, d))
        return v

    def half_argmax(val, ind):
        for d in (1, 2, 4):
            pidx = lax.bitwise_xor(iota, d)
            pv = _perm(val, pidx)
            pi = _perm(ind, pidx)
            take = jnp.logical_or(pv > val,
                                  jnp.logical_and(pv == val, pi < ind))
            val = jnp.where(take, pv, val)
            ind = jnp.where(take, pi, ind)
        return val, ind

    def chunk_body(c, _):
        tok0 = c * CHUNK
        pltpu.sync_copy(x_hbm.at[pl.ds(row0 + tok0, CHUNK)], x_v)

        def pair_body(p, carry):
            ci1, ci2, cp1, cp2 = carry
            t0 = 2 * p

            def k_body(kc, accs):
                ks = pl.ds(kc * LANES, LANES)
                xa = x_v[t0, ks]
                xb = x_v[t0 + 1, ks]
                new_a = []
                new_b = []
                for e in range(N_EXPERTS):
                    wv = w_v[e, ks]
                    new_a.append(accs[e] + xa * wv)
                    new_b.append(accs[N_EXPERTS + e] + xb * wv)
                return tuple(new_a + new_b)

            accs = lax.fori_loop(
                0, KCH, k_body,
                tuple(zero for _ in range(2 * N_EXPERTS)))

            v = butterfly_sum16(list(accs))
            lg_v[pl.ds((tok0 + t0) * N_EXPERTS, 2 * N_EXPERTS)] = v

            m1, i1 = half_argmax(v, iota8)
            masked = jnp.where(iota8 == i1, neg_inf, v)
            m2, i2 = half_argmax(masked, iota8)

            t = jnp.exp(m2 - m1)
            d2 = 1.0 + t
            p1 = 1.0 / d2
            p2 = t / d2

            ex = jnp.exp(v - m1)
            gp = ex / half_sum(ex)
            cnt = (jnp.where(iota8 == i1, 1.0, 0.0)
                   + jnp.where(iota8 == i2, 1.0, 0.0))
            acc_v[0, :] = acc_v[0, :] + cnt
            acc_v[1, :] = acc_v[1, :] + gp

            mask = jnp.logical_or(iota == t0, iota == t0 + 1)
            ci1 = jnp.where(mask, _perm(i1, cperm), ci1)
            ci2 = jnp.where(mask, _perm(i2, cperm), ci2)
            cp1 = jnp.where(mask, _perm(p1, cperm), cp1)
            cp2 = jnp.where(mask, _perm(p2, cperm), cp2)
            return ci1, ci2, cp1, cp2

        izero = jnp.zeros((LANES,), jnp.int32)
        ci1, ci2, cp1, cp2 = lax.fori_loop(
            0, CHUNK // 2, pair_body, (izero, izero, zero, zero))
        ts = pl.ds(tok0, CHUNK)
        i1_v[ts] = ci1
        i2_v[ts] = ci2
        p1_v[ts] = cp1
        p2_v[ts] = cp2
        return 0

    lax.fori_loop(0, NT // CHUNK, chunk_body, 0)

    out_tok = pl.ds(wid * NT, NT)
    pltpu.sync_copy(lg_v, lg_hbm.at[pl.ds(wid * NT * N_EXPERTS, NT * N_EXPERTS)])
    pltpu.sync_copy(i1_v, i1_hbm.at[out_tok])
    pltpu.sync_copy(i2_v, i2_hbm.at[out_tok])
    pltpu.sync_copy(p1_v, p1_hbm.at[out_tok])
    pltpu.sync_copy(p2_v, p2_hbm.at[out_tok])
    pltpu.sync_copy(acc_v, part_hbm.at[wid])


def _sc_router(x_flat, gate_w):
    mesh = plsc.VectorSubcoreMesh(core_axis_name="c", subcore_axis_name="s")
    f = pl.kernel(
        _sc_body,
        out_type=[
            jax.ShapeDtypeStruct((N_SC * N_EXPERTS,), jnp.float32),
            jax.ShapeDtypeStruct((N_SC,), jnp.int32),
            jax.ShapeDtypeStruct((N_SC,), jnp.int32),
            jax.ShapeDtypeStruct((N_SC,), jnp.float32),
            jax.ShapeDtypeStruct((N_SC,), jnp.float32),
            jax.ShapeDtypeStruct((NUM_TILES, 2, LANES), jnp.float32),
        ],
        mesh=mesh,
        scratch_types=[
            pltpu.VMEM((N_EXPERTS, D_MODEL), jnp.float32),
            pltpu.VMEM((CHUNK, D_MODEL), jnp.float32),
            pltpu.VMEM((NT * N_EXPERTS,), jnp.float32),
            pltpu.VMEM((NT,), jnp.int32),
            pltpu.VMEM((NT,), jnp.int32),
            pltpu.VMEM((NT,), jnp.float32),
            pltpu.VMEM((NT,), jnp.float32),
            pltpu.VMEM((2, LANES), jnp.float32),
        ],
    )
    return f(x_flat, gate_w)


def kernel(x, gate_w):
    b, s, d = x.shape
    n_tokens = b * s
    n_tc = n_tokens - N_SC
    x_flat = x.reshape(n_tokens, d)
    wt = gate_w.T

    lt_tc, idx_tc, prb_tc, part_blocks = _tc_router(x_flat, wt, n_tc)
    lg_sc, i1_sc, i2_sc, p1_sc, p2_sc, part_sc = _sc_router(x_flat, gate_w)

    logits = jnp.concatenate([lt_tc.T, lg_sc.reshape(N_SC, N_EXPERTS)], axis=0)
    idx = jnp.concatenate([idx_tc.T, jnp.stack([i1_sc, i2_sc], axis=1)], axis=0)
    prb = jnp.concatenate([prb_tc.T, jnp.stack([p1_sc, p2_sc], axis=1)], axis=0)

    part_tc = part_blocks.sum(axis=0)
    sc = part_sc.reshape(NUM_TILES, 2, 2, N_EXPERTS).sum(axis=(0, 2))
    cnt = part_tc[:, 0] + sc[0]
    ps = part_tc[:, 1] + sc[1]
    frac = cnt / (n_tokens * TOP_K)
    avg = ps / n_tokens
    loss = jnp.sum(frac * avg) * (N_EXPERTS * BALANCE_LOSS_WEIGHT)

    capacity = max(int(b * s * TOP_K / N_EXPERTS * CAPACITY_FACTOR), 4)
    return (idx.astype(jnp.int64),
            prb,
            logits,
            loss,
            jnp.asarray(capacity, dtype=jnp.int32))
